# baseline (speedup 1.0000x reference)
import jax
import jax.numpy as jnp
from jax import lax
from jax.experimental import pallas as pl
from jax.experimental.pallas import tpu as pltpu
from jax.experimental.pallas import tpu_sc as plsc

NC = 2
NS = 16
NW = NC * NS
L = 16
D = 32
CH = 128


def _pack_body(x_ref, y_ref):
    xt = x_ref[...].T
    y_ref[...] = jnp.concatenate(
        [xt[0:128], xt[128:256], xt[256:384], xt[384:512]], axis=1)


def _pack(table):
    n = table.shape[0]
    nb = (n + 511) // 512
    return pl.pallas_call(
        _pack_body,
        grid=(nb,),
        in_specs=[pl.BlockSpec((32, 512), lambda g: (0, g))],
        out_specs=pl.BlockSpec((128, 128), lambda g: (g, 0)),
        out_shape=jax.ShapeDtypeStruct((nb * 128, 128), jnp.float32),
    )(table.T)


def _sc_body(C,
             ent_p, rel_p, tem_p,
             pos_h, pos_r, pos_tem, pos_t,
             neg_h, neg_r, neg_tem, neg_t,
             pos_out, neg_out,
             idx_s, row_s, gh, gt, gm, rel_v, out_v, sem):
    wid = lax.axis_index("s") * NC + lax.axis_index("c")
    base = wid * C
    nch = C // CH

    pltpu.sync_copy(rel_p, rel_v)

    for idx_hbm, out_hbm in (
        ((pos_h, pos_t, pos_tem, pos_r), pos_out),
        ((neg_h, neg_t, neg_tem, neg_r), neg_out),
    ):
        for ch in range(nch):
            off = base + ch * CH
            cps = [pltpu.async_copy(idx_hbm[t].at[pl.ds(off, CH)],
                                    idx_s.at[t], sem) for t in range(4)]
            for c in cps:
                c.wait()
            for t in range(3):
                for v in range(CH // L):
                    sl = pl.ds(v * L, L)
                    iv = idx_s[t, sl]
                    row_s[t, sl] = (
                        lax.shift_left(lax.shift_right_logical(iv, 9), 7)
                        + (iv & 127))
            cps = [
                pltpu.async_copy(ent_p.at[row_s.at[0]], gh, sem),
                pltpu.async_copy(ent_p.at[row_s.at[1]], gt, sem),
                pltpu.async_copy(tem_p.at[row_s.at[2]], gm, sem),
            ]
            for c in cps:
                c.wait()
            for g in range(CH // L):
                sl = pl.ds(g * L, L)
                slot = g * L + lax.iota(jnp.int32, L)
                ch_cb = (lax.shift_right_logical(idx_s[0, sl], 7) & 3) * D
                ct_cb = (lax.shift_right_logical(idx_s[1, sl], 7) & 3) * D
                cm_cb = (lax.shift_right_logical(idx_s[2, sl], 7) & 3) * D
                r16 = idx_s[3, sl]
                rrow = (lax.shift_left(lax.shift_right_logical(r16, 9), 7)
                        + (r16 & 127))
                rcb = (lax.shift_right_logical(r16, 7) & 3) * D

                def col_step(j, acc):
                    h = plsc.load_gather(gh, [slot, ch_cb + j])
                    t_ = plsc.load_gather(gt, [slot, ct_cb + j])
                    m = plsc.load_gather(gm, [slot, cm_cb + j])
                    r = plsc.load_gather(rel_v, [rrow, rcb + j])
                    return acc + jnp.abs(h + r + m - t_)

                acc = lax.fori_loop(0, D, col_step,
                                    jnp.zeros((L,), jnp.float32))
                out_v[pl.ds(ch * CH + g * L, L)] = acc
        pltpu.sync_copy(out_v, out_hbm.at[pl.ds(base, C)])


def kernel(pos_h, pos_t, pos_r, pos_tem, neg_h, neg_t, neg_r, neg_tem,
           ent_w, rel_w, tem_w):
    B = pos_h.shape[0]
    C = B // NW
    ent_p = _pack(ent_w)
    rel_p = _pack(rel_w)
    tem_p = _pack(tem_w)
    mesh = plsc.VectorSubcoreMesh(core_axis_name="c", subcore_axis_name="s")

    def body(*refs):
        _sc_body(C, *refs)

    f = pl.kernel(
        body,
        out_type=(jax.ShapeDtypeStruct((B,), jnp.float32),
                  jax.ShapeDtypeStruct((B,), jnp.float32)),
        mesh=mesh,
        scratch_types=[
            pltpu.VMEM((4, CH), jnp.int32),
            pltpu.VMEM((3, CH), jnp.int32),
            pltpu.VMEM((CH, 128), jnp.float32),
            pltpu.VMEM((CH, 128), jnp.float32),
            pltpu.VMEM((CH, 128), jnp.float32),
            pltpu.VMEM((rel_p.shape[0], 128), jnp.float32),
            pltpu.VMEM((C,), jnp.float32),
            pltpu.SemaphoreType.DMA,
        ],
        compiler_params=pltpu.CompilerParams(needs_layout_passes=False),
    )
    i32 = jnp.int32
    return f(ent_p, rel_p, tem_p,
             pos_h.astype(i32), pos_r.astype(i32), pos_tem.astype(i32),
             pos_t.astype(i32),
             neg_h.astype(i32), neg_r.astype(i32), neg_tem.astype(i32),
             neg_t.astype(i32))


# speedup vs baseline: 4.1162x; 4.1162x over previous
import jax
import jax.numpy as jnp
from jax import lax
from jax.experimental import pallas as pl
from jax.experimental.pallas import tpu as pltpu
from jax.experimental.pallas import tpu_sc as plsc

NC = 2
NS = 16
NW = NC * NS
L = 16
D = 32
CH = 128


_PCOLS = 4096


def _pack_body(x_ref, y_ref):
    x = x_ref[...]
    for q in range(x.shape[1] // 512):
        off = 512 * q
        w = jnp.concatenate(
            [x[:, off + 128 * k:off + 128 * (k + 1)] for k in range(4)],
            axis=0)
        y_ref[pl.ds(128 * q, 128), :] = w.T


def _pack(table):
    n = table.shape[0]
    pcols = min(_PCOLS, ((n + 511) // 512) * 512)
    nb = (n + pcols - 1) // pcols
    return pl.pallas_call(
        _pack_body,
        grid=(nb,),
        in_specs=[pl.BlockSpec((32, pcols), lambda g: (0, g))],
        out_specs=pl.BlockSpec((pcols // 4, 128), lambda g: (g, 0)),
        out_shape=jax.ShapeDtypeStruct((nb * (pcols // 4), 128),
                                       jnp.float32),
    )(table.T)


def _sc_body(C,
             ent_p, rel_p, tem_p,
             pos_h, pos_r, pos_tem, pos_t,
             neg_h, neg_r, neg_tem, neg_t,
             pos_out, neg_out,
             idx_s, row_s, gh, gt, gm, rel_v, out_v, sem):
    wid = lax.axis_index("s") * NC + lax.axis_index("c")
    base = wid * C
    nch = C // CH

    pltpu.sync_copy(rel_p, rel_v)

    for idx_hbm, out_hbm in (
        ((pos_h, pos_t, pos_tem, pos_r), pos_out),
        ((neg_h, neg_t, neg_tem, neg_r), neg_out),
    ):
        for ch in range(nch):
            off = base + ch * CH
            cps = [pltpu.async_copy(idx_hbm[t].at[pl.ds(off, CH)],
                                    idx_s.at[t], sem) for t in range(4)]
            for c in cps:
                c.wait()
            for t in range(3):
                for v in range(CH // L):
                    sl = pl.ds(v * L, L)
                    iv = idx_s[t, sl]
                    row_s[t, sl] = (
                        lax.shift_left(lax.shift_right_logical(iv, 9), 7)
                        + (iv & 127))
            cps = [
                pltpu.async_copy(ent_p.at[row_s.at[0]], gh, sem),
                pltpu.async_copy(ent_p.at[row_s.at[1]], gt, sem),
                pltpu.async_copy(tem_p.at[row_s.at[2]], gm, sem),
            ]
            for c in cps:
                c.wait()
            for g in range(CH // L):
                sl = pl.ds(g * L, L)
                slot = g * L + lax.iota(jnp.int32, L)
                ch_cb = (lax.shift_right_logical(idx_s[0, sl], 7) & 3) * D
                ct_cb = (lax.shift_right_logical(idx_s[1, sl], 7) & 3) * D
                cm_cb = (lax.shift_right_logical(idx_s[2, sl], 7) & 3) * D
                r16 = idx_s[3, sl]
                rrow = (lax.shift_left(lax.shift_right_logical(r16, 9), 7)
                        + (r16 & 127))
                rcb = (lax.shift_right_logical(r16, 7) & 3) * D

                def col_step(j, acc):
                    h = plsc.load_gather(gh, [slot, ch_cb + j])
                    t_ = plsc.load_gather(gt, [slot, ct_cb + j])
                    m = plsc.load_gather(gm, [slot, cm_cb + j])
                    r = plsc.load_gather(rel_v, [rrow, rcb + j])
                    return acc + jnp.abs(h + r + m - t_)

                acc = lax.fori_loop(0, D, col_step,
                                    jnp.zeros((L,), jnp.float32))
                out_v[pl.ds(ch * CH + g * L, L)] = acc
        pltpu.sync_copy(out_v, out_hbm.at[pl.ds(base, C)])


def kernel(pos_h, pos_t, pos_r, pos_tem, neg_h, neg_t, neg_r, neg_tem,
           ent_w, rel_w, tem_w):
    B = pos_h.shape[0]
    C = B // NW
    ent_p = _pack(ent_w)
    rel_p = _pack(rel_w)
    tem_p = _pack(tem_w)
    mesh = plsc.VectorSubcoreMesh(core_axis_name="c", subcore_axis_name="s")

    def body(*refs):
        _sc_body(C, *refs)

    f = pl.kernel(
        body,
        out_type=(jax.ShapeDtypeStruct((B,), jnp.float32),
                  jax.ShapeDtypeStruct((B,), jnp.float32)),
        mesh=mesh,
        scratch_types=[
            pltpu.VMEM((4, CH), jnp.int32),
            pltpu.VMEM((3, CH), jnp.int32),
            pltpu.VMEM((CH, 128), jnp.float32),
            pltpu.VMEM((CH, 128), jnp.float32),
            pltpu.VMEM((CH, 128), jnp.float32),
            pltpu.VMEM((rel_p.shape[0], 128), jnp.float32),
            pltpu.VMEM((C,), jnp.float32),
            pltpu.SemaphoreType.DMA,
        ],
        compiler_params=pltpu.CompilerParams(needs_layout_passes=False),
    )
    i32 = jnp.int32
    return f(ent_p, rel_p, tem_p,
             pos_h.astype(i32), pos_r.astype(i32), pos_tem.astype(i32),
             pos_t.astype(i32),
             neg_h.astype(i32), neg_r.astype(i32), neg_tem.astype(i32),
             neg_t.astype(i32))


# speedup vs baseline: 5.7875x; 1.4060x over previous
import jax
import jax.numpy as jnp
from jax import lax
from jax.experimental import pallas as pl
from jax.experimental.pallas import tpu as pltpu
from jax.experimental.pallas import tpu_sc as plsc

NC = 2
NS = 16
NW = NC * NS
L = 16
D = 32
CH = 64
_PCOLS = 8192


def _pack_body(x_ref, y_ref):
    x = x_ref[...]
    for q in range(x.shape[1] // 512):
        off = 512 * q
        w = jnp.concatenate(
            [x[:, off + 128 * k:off + 128 * (k + 1)] for k in range(4)],
            axis=0)
        y_ref[pl.ds(128 * q, 128), :] = w.T


def _pack(table):
    n = table.shape[0]
    pcols = min(_PCOLS, ((n + 511) // 512) * 512)
    nb = (n + pcols - 1) // pcols
    return pl.pallas_call(
        _pack_body,
        grid=(nb,),
        in_specs=[pl.BlockSpec((32, pcols), lambda g: (0, g))],
        out_specs=pl.BlockSpec((pcols // 4, 128), lambda g: (g, 0)),
        out_shape=jax.ShapeDtypeStruct((nb * (pcols // 4), 128),
                                       jnp.float32),
    )(table.T)


def _prow(iv):
    return (lax.shift_left(lax.shift_right_logical(iv, 9), 7) + (iv & 127))


def _sc_body(C,
             ent_p, rel_p, tem_p,
             pos_h, pos_r, pos_tem, pos_t,
             neg_h, neg_r, neg_tem, neg_t,
             pos_out, neg_out,
             idx_s, row_s, g0, g1, rel_v, out_v, sem0, sem1, sem2):
    wid = lax.axis_index("s") * NC + lax.axis_index("c")
    base = wid * C
    nch = (2 * C) // CH
    cps = C // CH

    idx_in = (pos_h, pos_t, pos_tem, pos_r, neg_h, neg_t, neg_tem, neg_r)
    stg = [pltpu.async_copy(idx_in[t].at[pl.ds(base, C)], idx_s.at[t], sem2)
           for t in range(8)]
    for c in stg:
        c.wait()
    for t in range(6):
        src = (t // 3) * 4 + (t % 3)
        def rows_step(v, _, t=t, src=src):
            sl = pl.ds(v * L, L)
            row_s[t, sl] = _prow(idx_s[src, sl])
            return 0
        lax.fori_loop(0, C // L, rows_step, 0)

    tabs = (ent_p, ent_p, tem_p)
    sems = (sem0, sem1)
    bufs = (g0, g1)

    def fire(c):
        side, cc = c // cps, c % cps
        b = bufs[c % 2]
        return [pltpu.async_copy(
            tabs[t].at[row_s.at[3 * side + t, pl.ds(cc * CH, CH)]],
            b[t], sems[c % 2]) for t in range(3)]

    relcp = pltpu.async_copy(rel_p, rel_v, sem2)
    relcp.wait()
    inflight = fire(0)

    for c in range(nch):
        nxt = fire(c + 1) if c + 1 < nch else []
        for cp in inflight:
            cp.wait()
        inflight = nxt
        b = bufs[c % 2]
        side, cc = c // cps, c % cps
        ioff = side * 4
        soff = cc * CH

        def group(g, _):
            slot = g * L + lax.iota(jnp.int32, L)
            sl = pl.ds(soff + g * L, L)
            ch_cb = (lax.shift_right_logical(idx_s[ioff + 0, sl], 7) & 3) * D
            ct_cb = (lax.shift_right_logical(idx_s[ioff + 1, sl], 7) & 3) * D
            cm_cb = (lax.shift_right_logical(idx_s[ioff + 2, sl], 7) & 3) * D
            r16 = idx_s[ioff + 3, sl]
            rrow = _prow(r16)
            rcb = (lax.shift_right_logical(r16, 7) & 3) * D

            def oct_step(o, accs):
                a0, a1 = accs
                j0 = o * 8
                for jj in range(8):
                    j = j0 + jj
                    h = plsc.load_gather(b[0], [slot, ch_cb + j])
                    t_ = plsc.load_gather(b[1], [slot, ct_cb + j])
                    m = plsc.load_gather(b[2], [slot, cm_cb + j])
                    r = plsc.load_gather(rel_v, [rrow, rcb + j])
                    v = jnp.abs(h + r + m - t_)
                    if jj % 2 == 0:
                        a0 = a0 + v
                    else:
                        a1 = a1 + v
                return (a0, a1)

            z = jnp.zeros((L,), jnp.float32)
            a0, a1 = lax.fori_loop(0, D // 8, oct_step, (z, z))
            out_v[pl.ds(c * CH + g * L, L)] = a0 + a1
            return 0

        lax.fori_loop(0, CH // L, group, 0)

    pltpu.sync_copy(out_v.at[pl.ds(0, C)], pos_out.at[pl.ds(base, C)])
    pltpu.sync_copy(out_v.at[pl.ds(C, C)], neg_out.at[pl.ds(base, C)])


def kernel(pos_h, pos_t, pos_r, pos_tem, neg_h, neg_t, neg_r, neg_tem,
           ent_w, rel_w, tem_w):
    B = pos_h.shape[0]
    C = B // NW
    ent_p = _pack(ent_w)
    rel_p = _pack(rel_w)
    tem_p = _pack(tem_w)
    mesh = plsc.VectorSubcoreMesh(core_axis_name="c", subcore_axis_name="s")

    def body(*refs):
        _sc_body(C, *refs)

    gset = lambda: tuple(pltpu.VMEM((CH, 128), jnp.float32)
                         for _ in range(3))
    f = pl.kernel(
        body,
        out_type=(jax.ShapeDtypeStruct((B,), jnp.float32),
                  jax.ShapeDtypeStruct((B,), jnp.float32)),
        mesh=mesh,
        scratch_types=[
            pltpu.VMEM((8, C), jnp.int32),
            pltpu.VMEM((6, C), jnp.int32),
            gset(),
            gset(),
            pltpu.VMEM((rel_p.shape[0], 128), jnp.float32),
            pltpu.VMEM((2 * C,), jnp.float32),
            pltpu.SemaphoreType.DMA,
            pltpu.SemaphoreType.DMA,
            pltpu.SemaphoreType.DMA,
        ],
        compiler_params=pltpu.CompilerParams(needs_layout_passes=False),
    )
    i32 = jnp.int32
    return f(ent_p, rel_p, tem_p,
             pos_h.astype(i32), pos_r.astype(i32), pos_tem.astype(i32),
             pos_t.astype(i32),
             neg_h.astype(i32), neg_r.astype(i32), neg_tem.astype(i32),
             neg_t.astype(i32))


# speedup vs baseline: 6.7391x; 1.1644x over previous
import jax
import jax.numpy as jnp
from jax import lax
from jax.experimental import pallas as pl
from jax.experimental.pallas import tpu as pltpu
from jax.experimental.pallas import tpu_sc as plsc

NC = 2
NS = 16
NW = NC * NS
L = 16
D = 32
CH = 64
_PCOLS = 16384
NBUF = 3


def _pack_body(x_ref, y_ref):
    x = x_ref[...]
    for q in range(x.shape[1] // 512):
        off = 512 * q
        w = jnp.concatenate(
            [x[:, off + 128 * k:off + 128 * (k + 1)] for k in range(4)],
            axis=0)
        y_ref[pl.ds(128 * q, 128), :] = w.T


def _pack(table):
    n = table.shape[0]
    pcols = min(_PCOLS, ((n + 511) // 512) * 512)
    nb = (n + pcols - 1) // pcols
    return pl.pallas_call(
        _pack_body,
        grid=(nb,),
        in_specs=[pl.BlockSpec((32, pcols), lambda g: (0, g))],
        out_specs=pl.BlockSpec((pcols // 4, 128), lambda g: (g, 0)),
        out_shape=jax.ShapeDtypeStruct((nb * (pcols // 4), 128),
                                       jnp.float32),
    )(table.T)


def _prow(iv):
    return (lax.shift_left(lax.shift_right_logical(iv, 9), 7) + (iv & 127))


def _sc_body(C,
             ent_p, rel_p, tem_p,
             pos_h, pos_r, pos_tem, pos_t,
             neg_h, neg_r, neg_tem, neg_t,
             pos_out, neg_out,
             idx_s, row_s, cb_s, g0, g1, g2, rel_v, out_v,
             sem0, sem1, sem2, sem3):
    wid = lax.axis_index("s") * NC + lax.axis_index("c")
    base = wid * C
    nch = (2 * C) // CH
    cps = C // CH

    idx_in = (pos_h, pos_t, pos_tem, pos_r, neg_h, neg_t, neg_tem, neg_r)
    stg = [pltpu.async_copy(idx_in[t].at[pl.ds(base, C)], idx_s.at[t], sem3)
           for t in range(8)]
    relcp = pltpu.async_copy(rel_p, rel_v, sem3)
    for c in stg:
        c.wait()
    for t in range(8):
        side, tt = t // 4, t % 4
        src = side * 4 + tt
        def rows_step(v, _, t=t, src=src, tt=tt, side=side):
            sl = pl.ds(v * L, L)
            iv = idx_s[src, sl]
            cb_s[t, sl] = (lax.shift_right_logical(iv, 7) & 3) * D
            if tt < 3:
                row_s[3 * side + tt, sl] = _prow(iv)
            else:
                row_s[6 + side, sl] = _prow(iv)
            return 0
        lax.fori_loop(0, C // L, rows_step, 0)

    tabs = (ent_p, ent_p, tem_p)
    sems = (sem0, sem1, sem2)
    bufs = (g0, g1, g2)

    def fire(c):
        side, cc = c // cps, c % cps
        b = bufs[c % NBUF]
        return [pltpu.async_copy(
            tabs[t].at[row_s.at[3 * side + t, pl.ds(cc * CH, CH)]],
            b[t], sems[c % NBUF]) for t in range(3)]

    relcp.wait()
    inflight = [fire(c) for c in range(NBUF - 1)]

    for c in range(nch):
        inflight.append(fire(c + NBUF - 1) if c + NBUF - 1 < nch else [])
        for cp in inflight.pop(0):
            cp.wait()
        b = bufs[c % NBUF]
        side, cc = c // cps, c % cps
        ioff = side * 4
        soff = cc * CH

        def group(g, _):
            slot = g * L + lax.iota(jnp.int32, L)
            sl = pl.ds(soff + g * L, L)
            ch_cb = cb_s[ioff + 0, sl]
            ct_cb = cb_s[ioff + 1, sl]
            cm_cb = cb_s[ioff + 2, sl]
            rcb = cb_s[ioff + 3, sl]
            rrow = row_s[6 + side, sl]

            def hex_step(o, accs):
                a0, a1, a2, a3 = accs
                j0 = o * 16
                for jj in range(16):
                    j = j0 + jj
                    h = plsc.load_gather(b[0], [slot, ch_cb + j])
                    t_ = plsc.load_gather(b[1], [slot, ct_cb + j])
                    m = plsc.load_gather(b[2], [slot, cm_cb + j])
                    r = plsc.load_gather(rel_v, [rrow, rcb + j])
                    v = jnp.abs(h + r + m - t_)
                    if jj % 4 == 0:
                        a0 = a0 + v
                    elif jj % 4 == 1:
                        a1 = a1 + v
                    elif jj % 4 == 2:
                        a2 = a2 + v
                    else:
                        a3 = a3 + v
                return (a0, a1, a2, a3)

            z = jnp.zeros((L,), jnp.float32)
            a0, a1, a2, a3 = lax.fori_loop(0, D // 16, hex_step,
                                           (z, z, z, z))
            out_v[pl.ds(c * CH + g * L, L)] = (a0 + a1) + (a2 + a3)
            return 0

        lax.fori_loop(0, CH // L, group, 0)

    pltpu.sync_copy(out_v.at[pl.ds(0, C)], pos_out.at[pl.ds(base, C)])
    pltpu.sync_copy(out_v.at[pl.ds(C, C)], neg_out.at[pl.ds(base, C)])


def kernel(pos_h, pos_t, pos_r, pos_tem, neg_h, neg_t, neg_r, neg_tem,
           ent_w, rel_w, tem_w):
    B = pos_h.shape[0]
    C = B // NW
    ent_p = _pack(ent_w)
    rel_p = _pack(rel_w)
    tem_p = _pack(tem_w)
    mesh = plsc.VectorSubcoreMesh(core_axis_name="c", subcore_axis_name="s")

    def body(*refs):
        _sc_body(C, *refs)

    gset = lambda: tuple(pltpu.VMEM((CH, 128), jnp.float32)
                         for _ in range(3))
    f = pl.kernel(
        body,
        out_type=(jax.ShapeDtypeStruct((B,), jnp.float32),
                  jax.ShapeDtypeStruct((B,), jnp.float32)),
        mesh=mesh,
        scratch_types=[
            pltpu.VMEM((8, C), jnp.int32),
            pltpu.VMEM((8, C), jnp.int32),
            pltpu.VMEM((8, C), jnp.int32),
            gset(),
            gset(),
            gset(),
            pltpu.VMEM((rel_p.shape[0], 128), jnp.float32),
            pltpu.VMEM((2 * C,), jnp.float32),
            pltpu.SemaphoreType.DMA,
            pltpu.SemaphoreType.DMA,
            pltpu.SemaphoreType.DMA,
            pltpu.SemaphoreType.DMA,
        ],
        compiler_params=pltpu.CompilerParams(needs_layout_passes=False),
    )
    i32 = jnp.int32
    return f(ent_p, rel_p, tem_p,
             pos_h.astype(i32), pos_r.astype(i32), pos_tem.astype(i32),
             pos_t.astype(i32),
             neg_h.astype(i32), neg_r.astype(i32), neg_tem.astype(i32),
             neg_t.astype(i32))


# speedup vs baseline: 9.6096x; 1.4259x over previous
import jax
import jax.numpy as jnp
from jax import lax
from jax.experimental import pallas as pl
from jax.experimental.pallas import tpu as pltpu
from jax.experimental.pallas import tpu_sc as plsc

NC = 2
NS = 16
NW = NC * NS
L = 16
D = 32
CH = 64
_PCOLS = 32768
NBUF = 3


def _pack_body(x_ref, y_ref):
    x = x_ref[...]
    for q in range(x.shape[1] // 512):
        off = 512 * q
        w = jnp.concatenate(
            [x[:, off + 128 * k:off + 128 * (k + 1)] for k in range(4)],
            axis=0)
        y_ref[pl.ds(128 * q, 128), :] = w.T


def _pack(table):
    n = table.shape[0]
    pcols = min(_PCOLS, ((n + 511) // 512) * 512)
    nb = (n + pcols - 1) // pcols
    return pl.pallas_call(
        _pack_body,
        grid=(nb,),
        in_specs=[pl.BlockSpec((32, pcols), lambda g: (0, g))],
        out_specs=pl.BlockSpec((pcols // 4, 128), lambda g: (g, 0)),
        out_shape=jax.ShapeDtypeStruct((nb * (pcols // 4), 128),
                                       jnp.float32),
    )(table.T)


def _prow(iv):
    return (lax.shift_left(lax.shift_right_logical(iv, 9), 7) + (iv & 127))


def _sc_body(C,
             ent_p, rel_p, tem_p,
             pos_h, pos_r, pos_tem, pos_t,
             neg_h, neg_r, neg_tem, neg_t,
             pos_out, neg_out,
             idx_s, row_s, cb_s, g0, g1, g2, rel_v, out_v,
             sem0, sem1, sem2, sem3):
    wid = lax.axis_index("s") * NC + lax.axis_index("c")
    base = wid * C
    nch = (2 * C) // CH
    cps = C // CH

    idx_in = (pos_h, pos_t, pos_tem, pos_r, neg_h, neg_t, neg_tem, neg_r)
    stg = [pltpu.async_copy(idx_in[t].at[pl.ds(base, C)], idx_s.at[t], sem3)
           for t in range(8)]
    relcp = pltpu.async_copy(rel_p, rel_v, sem3)
    for c in stg:
        c.wait()
    for t in range(8):
        side, tt = t // 4, t % 4
        src = side * 4 + tt
        def rows_step(v, _, t=t, src=src, tt=tt, side=side):
            sl = pl.ds(v * L, L)
            iv = idx_s[src, sl]
            cb_s[t, sl] = (lax.shift_right_logical(iv, 7) & 3) * D
            if tt < 3:
                row_s[3 * side + tt, sl] = _prow(iv)
            else:
                row_s[6 + side, sl] = _prow(iv)
            return 0
        lax.fori_loop(0, C // L, rows_step, 0)

    tabs = (ent_p, ent_p, tem_p)
    sems = (sem0, sem1, sem2)
    bufs = (g0, g1, g2)

    def fire(c):
        side, cc = c // cps, c % cps
        b = bufs[c % NBUF]
        return [pltpu.async_copy(
            tabs[t].at[row_s.at[3 * side + t, pl.ds(cc * CH, CH)]],
            b[t], sems[c % NBUF]) for t in range(3)]

    relcp.wait()
    inflight = [fire(c) for c in range(NBUF - 1)]

    for c in range(nch):
        inflight.append(fire(c + NBUF - 1) if c + NBUF - 1 < nch else [])
        for cp in inflight.pop(0):
            cp.wait()
        b = bufs[c % NBUF]
        side, cc = c // cps, c % cps
        ioff = side * 4
        soff = cc * CH

        def group(g, _):
            slot = g * L + lax.iota(jnp.int32, L)
            sl = pl.ds(soff + g * L, L)
            ch_cb = cb_s[ioff + 0, sl]
            ct_cb = cb_s[ioff + 1, sl]
            cm_cb = cb_s[ioff + 2, sl]
            rcb = cb_s[ioff + 3, sl]
            rrow = row_s[6 + side, sl]

            lane = lax.iota(jnp.int32, L)

            def hex_step(o, accs):
                a0, a1, a2, a3 = accs
                j0 = o * 16
                for jj in range(16):
                    jc = (j0 + jj + lane) & (D - 1)
                    h = plsc.load_gather(b[0], [slot, ch_cb + jc])
                    t_ = plsc.load_gather(b[1], [slot, ct_cb + jc])
                    m = plsc.load_gather(b[2], [slot, cm_cb + jc])
                    r = plsc.load_gather(rel_v, [rrow, rcb + jc])
                    v = jnp.abs(h + r + m - t_)
                    if jj % 4 == 0:
                        a0 = a0 + v
                    elif jj % 4 == 1:
                        a1 = a1 + v
                    elif jj % 4 == 2:
                        a2 = a2 + v
                    else:
                        a3 = a3 + v
                return (a0, a1, a2, a3)

            z = jnp.zeros((L,), jnp.float32)
            a0, a1, a2, a3 = lax.fori_loop(0, D // 16, hex_step,
                                           (z, z, z, z))
            out_v[pl.ds(c * CH + g * L, L)] = (a0 + a1) + (a2 + a3)
            return 0

        lax.fori_loop(0, CH // L, group, 0)

    pltpu.sync_copy(out_v.at[pl.ds(0, C)], pos_out.at[pl.ds(base, C)])
    pltpu.sync_copy(out_v.at[pl.ds(C, C)], neg_out.at[pl.ds(base, C)])


def kernel(pos_h, pos_t, pos_r, pos_tem, neg_h, neg_t, neg_r, neg_tem,
           ent_w, rel_w, tem_w):
    B = pos_h.shape[0]
    C = B // NW
    ent_p = _pack(ent_w)
    rel_p = _pack(rel_w)
    tem_p = _pack(tem_w)
    mesh = plsc.VectorSubcoreMesh(core_axis_name="c", subcore_axis_name="s")

    def body(*refs):
        _sc_body(C, *refs)

    gset = lambda: tuple(pltpu.VMEM((CH, 128), jnp.float32)
                         for _ in range(3))
    f = pl.kernel(
        body,
        out_type=(jax.ShapeDtypeStruct((B,), jnp.float32),
                  jax.ShapeDtypeStruct((B,), jnp.float32)),
        mesh=mesh,
        scratch_types=[
            pltpu.VMEM((8, C), jnp.int32),
            pltpu.VMEM((8, C), jnp.int32),
            pltpu.VMEM((8, C), jnp.int32),
            gset(),
            gset(),
            gset(),
            pltpu.VMEM((rel_p.shape[0], 128), jnp.float32),
            pltpu.VMEM((2 * C,), jnp.float32),
            pltpu.SemaphoreType.DMA,
            pltpu.SemaphoreType.DMA,
            pltpu.SemaphoreType.DMA,
            pltpu.SemaphoreType.DMA,
        ],
        compiler_params=pltpu.CompilerParams(needs_layout_passes=False),
    )
    i32 = jnp.int32
    return f(ent_p, rel_p, tem_p,
             pos_h.astype(i32), pos_r.astype(i32), pos_tem.astype(i32),
             pos_t.astype(i32),
             neg_h.astype(i32), neg_r.astype(i32), neg_tem.astype(i32),
             neg_t.astype(i32))
